# baseline (speedup 1.0000x reference)
import functools

import jax
import jax.numpy as jnp
from jax import lax
from jax.experimental import pallas as pl
from jax.experimental.pallas import tpu as pltpu
from jax.experimental.pallas import tpu_sc as plsc

_NC, _NS = 2, 16
_NW = _NC * _NS
_CHUNK = 128


def _sc_gather_two(user_pairs, user_pidx2d, item_pairs, item_pidx2d, B):
    b_per_w = B // _NW
    n_chunks = b_per_w // _CHUNK
    mesh = plsc.VectorSubcoreMesh(core_axis_name="c", subcore_axis_name="s")

    @functools.partial(
        pl.kernel,
        mesh=mesh,
        out_type=(
            jax.ShapeDtypeStruct((B, 128), jnp.float32),
            jax.ShapeDtypeStruct((B, 128), jnp.float32),
        ),
        scratch_types=[
            pltpu.VMEM((n_chunks, _CHUNK), jnp.int32),
            pltpu.VMEM((b_per_w, 128), jnp.float32),
            pltpu.SemaphoreType.DMA,
        ],
    )
    def k(ut_hbm, ui_hbm, it_hbm, ii_hbm, ou_hbm, oi_hbm,
          idx_v, rows_v, sem):
        wid = lax.axis_index("s") * _NC + lax.axis_index("c")
        base = wid * b_per_w
        row0 = wid * n_chunks

        def one_table(tab_hbm, pidx_hbm, out_hbm):
            pltpu.sync_copy(pidx_hbm.at[pl.ds(row0, n_chunks)], idx_v)
            handles = []
            for j in range(n_chunks):
                handles.append(pltpu.async_copy(
                    tab_hbm.at[idx_v.at[j]],
                    rows_v.at[pl.ds(j * _CHUNK, _CHUNK)], sem))
            for h in handles:
                h.wait()
            pltpu.sync_copy(rows_v, out_hbm.at[pl.ds(base, b_per_w)])

        one_table(ut_hbm, ui_hbm, ou_hbm)
        one_table(it_hbm, ii_hbm, oi_hbm)

    return k(user_pairs, user_pidx2d, item_pairs, item_pidx2d)


def _mlp_body(uep, iep, upar, ipar, uf, itf,
              ufW1, ufb1, ufW2, ufb2, ifW1, ifb1, ifW2, ifb2,
              W0u, W0i, W0uf, W0if, b0, W1, b1, W2, b2, W3, b3,
              out_ref):
    zero = jnp.float32(0.0)
    up = uep[...]
    ue = jnp.where(upar[...] > zero, up[:, 64:128], up[:, 0:64])
    ip = iep[...]
    ie = jnp.where(ipar[...] > zero, ip[:, 64:128], ip[:, 0:64])
    u = jnp.maximum(uf[...] @ ufW1[...] + ufb1[...], zero)
    u = jnp.maximum(u @ ufW2[...] + ufb2[...], zero)
    v = jnp.maximum(itf[...] @ ifW1[...] + ifb1[...], zero)
    v = jnp.maximum(v @ ifW2[...] + ifb2[...], zero)
    h = (ue @ W0u[...] + ie @ W0i[...]
         + u @ W0uf[...] + v @ W0if[...] + b0[...])
    h = jnp.maximum(h, zero)
    h = jnp.maximum(h @ W1[...] + b1[...], zero)
    h = jnp.maximum(h @ W2[...] + b2[...], zero)
    out_ref[...] = h @ W3[...] + b3[...]


def kernel(user_ids, item_ids, user_features, item_features, params):
    p = params
    B = user_ids.shape[0]
    D = p['user_table'].shape[1]
    eps = 1e-5

    user_pairs = p['user_table'].reshape(-1, 2 * D)
    item_pairs = p['item_table'].reshape(-1, 2 * D)
    user_pidx2d = (user_ids >> 1).reshape(B // _CHUNK, _CHUNK)
    item_pidx2d = (item_ids >> 1).reshape(B // _CHUNK, _CHUNK)
    upar = (user_ids & 1).astype(jnp.float32).reshape(B, 1)
    ipar = (item_ids & 1).astype(jnp.float32).reshape(B, 1)
    user_emb, item_emb = _sc_gather_two(
        user_pairs, user_pidx2d, item_pairs, item_pidx2d, B)

    s0 = p['g0'] / jnp.sqrt(p['v0'] + eps)
    t0 = p['be0'] - p['m0'] * s0
    s1 = p['g1'] / jnp.sqrt(p['v1'] + eps)
    t1 = p['be1'] - p['m1'] * s1
    s2 = p['g2'] / jnp.sqrt(p['v2'] + eps)
    t2 = p['be2'] - p['m2'] * s2
    W1f = s0[:, None] * p['W1']
    b1f = p['b1'] + t0 @ p['W1']
    W2f = s1[:, None] * p['W2']
    b2f = p['b2'] + t1 @ p['W2']
    W3f = s2[:, None] * p['W3']
    b3f = p['b3'] + t2 @ p['W3']

    W0 = p['W0']
    W0u, W0i, W0uf, W0if = W0[0:D], W0[D:2 * D], W0[2 * D:3 * D], W0[3 * D:4 * D]

    BB = 2048
    row2d = lambda a: a.reshape(1, -1)
    full = lambda a: pl.BlockSpec(a.shape, lambda i: (0, 0))
    weights = [p['uf_W1'], row2d(p['uf_b1']), p['uf_W2'], row2d(p['uf_b2']),
               p['if_W1'], row2d(p['if_b1']), p['if_W2'], row2d(p['if_b2']),
               W0u, W0i, W0uf, W0if, row2d(p['b0']),
               W1f, row2d(b1f), W2f, row2d(b2f), W3f, row2d(b3f)]

    out = pl.pallas_call(
        _mlp_body,
        grid=(B // BB,),
        in_specs=[
            pl.BlockSpec((BB, 2 * D), lambda i: (i, 0)),
            pl.BlockSpec((BB, 2 * D), lambda i: (i, 0)),
            pl.BlockSpec((BB, 1), lambda i: (i, 0)),
            pl.BlockSpec((BB, 1), lambda i: (i, 0)),
            pl.BlockSpec((BB, user_features.shape[1]), lambda i: (i, 0)),
            pl.BlockSpec((BB, item_features.shape[1]), lambda i: (i, 0)),
        ] + [full(w) for w in weights],
        out_specs=pl.BlockSpec((BB, 1), lambda i: (i, 0)),
        out_shape=jax.ShapeDtypeStruct((B, 1), jnp.float32),
    )(user_emb, item_emb, upar, ipar, user_features, item_features, *weights)
    return out[:, 0]
